# Initial kernel scaffold; baseline (speedup 1.0000x reference)
import functools

import jax
import jax.numpy as jnp
from jax.experimental import pallas as pl


def _interp_body(bq_ref, q_ref, bs_ref, pt_ref, x_ref, *rest, k, n_src, affine):
    if affine:
        sc_ref, sh_ref, o_ref = rest
    else:
        (o_ref,) = rest
    q = q_ref[:]
    q0, q1, q2 = q[:, 0:1], q[:, 1:2], q[:, 2:3]
    p0 = pt_ref[0:1, :]
    p1 = pt_ref[1:2, :]
    p2 = pt_ref[2:3, :]
    qn = q0 * q0 + q1 * q1 + q2 * q2
    pn = p0 * p0 + p1 * p1 + p2 * p2
    cross = q0 * p0 + q1 * p1 + q2 * p2
    d2 = (qn - 2.0 * cross) + pn
    bmask = bq_ref[:] != bs_ref[:]
    d2 = jnp.where(bmask, 1e10, d2)

    iota = jax.lax.broadcasted_iota(jnp.int32, d2.shape, 1)
    S = None
    wsum = None
    for j in range(k):
        m = jnp.min(d2, axis=1, keepdims=True)
        am = jnp.min(jnp.where(d2 == m, iota, n_src), axis=1, keepdims=True)
        sel = iota == am
        if k == 1:
            S = sel.astype(jnp.float32)
        else:
            w = 1.0 / jnp.clip(m, 1e-16, None)
            contrib = jnp.where(sel, w, 0.0)
            S = contrib if S is None else S + contrib
            wsum = w if wsum is None else wsum + w
            if j + 1 < k:
                d2 = jnp.where(sel, 1e30, d2)

    out = jnp.dot(S, x_ref[:], preferred_element_type=jnp.float32)
    if k > 1:
        out = out / wsum
    if affine:
        out = out * sc_ref[:] + sh_ref[:]
    o_ref[:] = out


def _interp(x, pos_src_t, pos_q, bs_row, bq_col, k, affine=None, bq=256):
    n_src = pos_src_t.shape[1]
    n_q = pos_q.shape[0]
    c = x.shape[1]
    bq = min(bq, n_q)
    grid = (n_q // bq,)
    in_specs = [
        pl.BlockSpec((bq, 1), lambda i: (i, 0)),
        pl.BlockSpec((bq, 3), lambda i: (i, 0)),
        pl.BlockSpec((1, n_src), lambda i: (0, 0)),
        pl.BlockSpec((3, n_src), lambda i: (0, 0)),
        pl.BlockSpec((n_src, c), lambda i: (0, 0)),
    ]
    args = [bq_col, pos_q, bs_row, pos_src_t, x]
    if affine is not None:
        scale, shift = affine
        in_specs += [pl.BlockSpec((1, c), lambda i: (0, 0)),
                     pl.BlockSpec((1, c), lambda i: (0, 0))]
        args += [scale, shift]
    return pl.pallas_call(
        functools.partial(_interp_body, k=k, n_src=n_src,
                          affine=affine is not None),
        grid=grid,
        in_specs=in_specs,
        out_specs=pl.BlockSpec((bq, c), lambda i: (i, 0)),
        out_shape=jax.ShapeDtypeStruct((n_q, c), jnp.float32),
    )(*args)


def _linear_body(x_ref, w_ref, b_ref, *rest, affine):
    if affine:
        sc_ref, sh_ref, h_ref, s_ref, ss_ref = rest
    else:
        h_ref, s_ref, ss_ref = rest
    x = x_ref[:]
    if affine:
        x = x * sc_ref[:] + sh_ref[:]
    h = jnp.dot(x, w_ref[:], preferred_element_type=jnp.float32) + b_ref[:]
    h = jnp.maximum(h, 0.0)
    h_ref[:] = h
    csum = jnp.sum(h, axis=0, keepdims=True)
    csq = jnp.sum(h * h, axis=0, keepdims=True)

    @pl.when(pl.program_id(0) == 0)
    def _init():
        s_ref[:] = csum
        ss_ref[:] = csq

    @pl.when(pl.program_id(0) > 0)
    def _acc():
        s_ref[:] += csum
        ss_ref[:] += csq


def _linear(x, w, b_row, affine=None, br=512):
    n, fin = x.shape
    fout = w.shape[1]
    br = min(br, n)
    grid = (n // br,)
    in_specs = [
        pl.BlockSpec((br, fin), lambda i: (i, 0)),
        pl.BlockSpec((fin, fout), lambda i: (0, 0)),
        pl.BlockSpec((1, fout), lambda i: (0, 0)),
    ]
    args = [x, w, b_row]
    if affine is not None:
        scale, shift = affine
        in_specs += [pl.BlockSpec((1, fin), lambda i: (0, 0)),
                     pl.BlockSpec((1, fin), lambda i: (0, 0))]
        args += [scale, shift]
    return pl.pallas_call(
        functools.partial(_linear_body, affine=affine is not None),
        grid=grid,
        in_specs=in_specs,
        out_specs=[
            pl.BlockSpec((br, fout), lambda i: (i, 0)),
            pl.BlockSpec((1, fout), lambda i: (0, 0)),
            pl.BlockSpec((1, fout), lambda i: (0, 0)),
        ],
        out_shape=[
            jax.ShapeDtypeStruct((n, fout), jnp.float32),
            jax.ShapeDtypeStruct((1, fout), jnp.float32),
            jax.ShapeDtypeStruct((1, fout), jnp.float32),
        ],
    )(*args)


def _affine_body(s_ref, ss_ref, g_ref, be_ref, sc_ref, sh_ref, *, n):
    mean = s_ref[:] * (1.0 / n)
    var = ss_ref[:] * (1.0 / n) - mean * mean
    scale = g_ref[:] / jnp.sqrt(var + 1e-5)
    sc_ref[:] = scale
    sh_ref[:] = be_ref[:] - mean * scale


def _affine(s, ss, g, be, n):
    f = s.shape[1]
    spec = pl.BlockSpec((1, f), lambda: (0, 0))
    return pl.pallas_call(
        functools.partial(_affine_body, n=float(n)),
        in_specs=[spec, spec, spec, spec],
        out_specs=[spec, spec],
        out_shape=[jax.ShapeDtypeStruct((1, f), jnp.float32),
                   jax.ShapeDtypeStruct((1, f), jnp.float32)],
    )(s, ss, g.reshape(1, f), be.reshape(1, f))


def _bn_apply_body(h_ref, sc_ref, sh_ref, o_ref):
    o_ref[:] = h_ref[:] * sc_ref[:] + sh_ref[:]


def _bn_apply(h, affine, br=1024):
    n, f = h.shape
    scale, shift = affine
    grid = (n // br,)
    return pl.pallas_call(
        _bn_apply_body,
        grid=grid,
        in_specs=[pl.BlockSpec((br, f), lambda i: (i, 0)),
                  pl.BlockSpec((1, f), lambda i: (0, 0)),
                  pl.BlockSpec((1, f), lambda i: (0, 0))],
        out_specs=pl.BlockSpec((br, f), lambda i: (i, 0)),
        out_shape=jax.ShapeDtypeStruct((n, f), jnp.float32),
    )(h, scale, shift)


def kernel(x0, x1, x2, x3, p0, p1, p2, p3, b0, b1, b2, b3,
           l3w1, l3b1, l3g1, l3be1, l3w2, l3b2, l3g2, l3be2,
           l2w1, l2b1, l2g1, l2be1, l2w2, l2b2, l2g2, l2be2,
           l1w1, l1b1, l1g1, l1be1, l1w2, l1b2, l1g2, l1be2):
    b0c = b0.reshape(-1, 1)
    b1c = b1.reshape(-1, 1)
    b2c = b2.reshape(-1, 1)
    b1r = b1.reshape(1, -1)
    b2r = b2.reshape(1, -1)
    b3r = b3.reshape(1, -1)
    p1t = p1.T
    p2t = p2.T
    p3t = p3.T

    h = _interp(x3, p3t, p2, b3r, b2c, k=1)
    h = jnp.concatenate([h, x2], axis=1)
    h, s, ss = _linear(h, l3w1, l3b1.reshape(1, -1))
    a = _affine(s, ss, l3g1, l3be1, 1024)
    h, s, ss = _linear(h, l3w2, l3b2.reshape(1, -1), affine=a)
    a = _affine(s, ss, l3g2, l3be2, 1024)

    h = _interp(h, p2t, p1, b2r, b1c, k=3, affine=a)
    h = jnp.concatenate([h, x1], axis=1)
    h, s, ss = _linear(h, l2w1, l2b1.reshape(1, -1))
    a = _affine(s, ss, l2g1, l2be1, 4096)
    h, s, ss = _linear(h, l2w2, l2b2.reshape(1, -1), affine=a)
    a = _affine(s, ss, l2g2, l2be2, 4096)

    h = _interp(h, p1t, p0, b1r, b0c, k=3, affine=a)
    h = jnp.concatenate([h, x0], axis=1)
    h, s, ss = _linear(h, l1w1, l1b1.reshape(1, -1))
    a = _affine(s, ss, l1g1, l1be1, 16384)
    h, s, ss = _linear(h, l1w2, l1b2.reshape(1, -1), affine=a)
    a = _affine(s, ss, l1g2, l1be2, 16384)

    return _bn_apply(h, a)


# speedup vs baseline: 8.2963x; 8.2963x over previous
import functools

import jax
import jax.numpy as jnp
from jax.experimental import pallas as pl


def _interp_body(bq_ref, q_ref, bs_ref, pt_ref, x_ref, *rest, k, n_src, affine,
                 cross_bf16):
    if affine:
        sc_ref, sh_ref, o_ref = rest
    else:
        (o_ref,) = rest
    q = q_ref[:]
    q0, q1, q2 = q[:, 0:1], q[:, 1:2], q[:, 2:3]
    p0 = pt_ref[0:1, :]
    p1 = pt_ref[1:2, :]
    p2 = pt_ref[2:3, :]
    qn = q0 * q0 + q1 * q1 + q2 * q2
    pn = p0 * p0 + p1 * p1 + p2 * p2
    if cross_bf16:
        cross = jnp.dot(q.astype(jnp.bfloat16), pt_ref[:].astype(jnp.bfloat16),
                        preferred_element_type=jnp.float32)
    else:
        cross = q0 * p0 + q1 * p1 + q2 * p2
    d2 = (qn - 2.0 * cross) + pn
    bmask = bq_ref[:] != bs_ref[:]
    d2 = jnp.where(bmask, 1e10, d2)

    iota = jax.lax.broadcasted_iota(jnp.int32, d2.shape, 1)
    S = None
    wsum = None
    for j in range(k):
        m = jnp.min(d2, axis=1, keepdims=True)
        am = jnp.min(jnp.where(d2 == m, iota, n_src), axis=1, keepdims=True)
        sel = iota == am
        if k == 1:
            S = sel.astype(jnp.float32)
        else:
            w = 1.0 / jnp.clip(m, 1e-16, None)
            contrib = jnp.where(sel, w, 0.0)
            S = contrib if S is None else S + contrib
            wsum = w if wsum is None else wsum + w
            if j + 1 < k:
                d2 = jnp.where(sel, 1e30, d2)

    out = jnp.dot(S, x_ref[:], preferred_element_type=jnp.float32)
    if k > 1:
        out = out / wsum
    if affine:
        out = out * sc_ref[:] + sh_ref[:]
    o_ref[:] = out


def _interp(x, pos_src_t, pos_q, bs_row, bq_col, k, affine=None, bq=256,
            cross_bf16=False):
    n_src = pos_src_t.shape[1]
    n_q = pos_q.shape[0]
    c = x.shape[1]
    bq = min(bq, n_q)
    grid = (n_q // bq,)
    in_specs = [
        pl.BlockSpec((bq, 1), lambda i: (i, 0)),
        pl.BlockSpec((bq, 3), lambda i: (i, 0)),
        pl.BlockSpec((1, n_src), lambda i: (0, 0)),
        pl.BlockSpec((3, n_src), lambda i: (0, 0)),
        pl.BlockSpec((n_src, c), lambda i: (0, 0)),
    ]
    args = [bq_col, pos_q, bs_row, pos_src_t, x]
    if affine is not None:
        scale, shift = affine
        in_specs += [pl.BlockSpec((1, c), lambda i: (0, 0)),
                     pl.BlockSpec((1, c), lambda i: (0, 0))]
        args += [scale, shift]
    return pl.pallas_call(
        functools.partial(_interp_body, k=k, n_src=n_src,
                          affine=affine is not None, cross_bf16=cross_bf16),
        grid=grid,
        in_specs=in_specs,
        out_specs=pl.BlockSpec((bq, c), lambda i: (i, 0)),
        out_shape=jax.ShapeDtypeStruct((n_q, c), jnp.float32),
    )(*args)


def _linear_body(x_ref, w_ref, b_ref, *rest, affine, mm):
    if affine:
        sc_ref, sh_ref, h_ref, s_ref, ss_ref = rest
    else:
        h_ref, s_ref, ss_ref = rest
    x = x_ref[:]
    if affine:
        x = x * sc_ref[:] + sh_ref[:]
    if mm == "bf16":
        h = jnp.dot(x.astype(jnp.bfloat16), w_ref[:].astype(jnp.bfloat16),
                    preferred_element_type=jnp.float32)
    elif mm == "hi":
        h = jnp.dot(x, w_ref[:], precision=jax.lax.Precision.HIGHEST,
                    preferred_element_type=jnp.float32)
    else:
        h = jnp.dot(x, w_ref[:], preferred_element_type=jnp.float32)
    h = h + b_ref[:]
    h = jnp.maximum(h, 0.0)
    h_ref[:] = h
    csum = jnp.sum(h, axis=0, keepdims=True)
    csq = jnp.sum(h * h, axis=0, keepdims=True)

    @pl.when(pl.program_id(0) == 0)
    def _init():
        s_ref[:] = csum
        ss_ref[:] = csq

    @pl.when(pl.program_id(0) > 0)
    def _acc():
        s_ref[:] += csum
        ss_ref[:] += csq


def _linear(x, w, b_row, affine=None, br=512, mm="default"):
    n, fin = x.shape
    fout = w.shape[1]
    br = min(br, n)
    grid = (n // br,)
    in_specs = [
        pl.BlockSpec((br, fin), lambda i: (i, 0)),
        pl.BlockSpec((fin, fout), lambda i: (0, 0)),
        pl.BlockSpec((1, fout), lambda i: (0, 0)),
    ]
    args = [x, w, b_row]
    if affine is not None:
        scale, shift = affine
        in_specs += [pl.BlockSpec((1, fin), lambda i: (0, 0)),
                     pl.BlockSpec((1, fin), lambda i: (0, 0))]
        args += [scale, shift]
    return pl.pallas_call(
        functools.partial(_linear_body, affine=affine is not None, mm=mm),
        grid=grid,
        in_specs=in_specs,
        out_specs=[
            pl.BlockSpec((br, fout), lambda i: (i, 0)),
            pl.BlockSpec((1, fout), lambda i: (0, 0)),
            pl.BlockSpec((1, fout), lambda i: (0, 0)),
        ],
        out_shape=[
            jax.ShapeDtypeStruct((n, fout), jnp.float32),
            jax.ShapeDtypeStruct((1, fout), jnp.float32),
            jax.ShapeDtypeStruct((1, fout), jnp.float32),
        ],
    )(*args)


def _affine_body(s_ref, ss_ref, g_ref, be_ref, sc_ref, sh_ref, *, n):
    mean = s_ref[:] * (1.0 / n)
    var = ss_ref[:] * (1.0 / n) - mean * mean
    scale = g_ref[:] / jnp.sqrt(var + 1e-5)
    sc_ref[:] = scale
    sh_ref[:] = be_ref[:] - mean * scale


def _affine(s, ss, g, be, n):
    f = s.shape[1]
    spec = pl.BlockSpec((1, f), lambda: (0, 0))
    return pl.pallas_call(
        functools.partial(_affine_body, n=float(n)),
        in_specs=[spec, spec, spec, spec],
        out_specs=[spec, spec],
        out_shape=[jax.ShapeDtypeStruct((1, f), jnp.float32),
                   jax.ShapeDtypeStruct((1, f), jnp.float32)],
    )(s, ss, g.reshape(1, f), be.reshape(1, f))


def _bn_apply_body(h_ref, sc_ref, sh_ref, o_ref):
    o_ref[:] = h_ref[:] * sc_ref[:] + sh_ref[:]


def _bn_apply(h, affine, br=1024):
    n, f = h.shape
    scale, shift = affine
    grid = (n // br,)
    return pl.pallas_call(
        _bn_apply_body,
        grid=grid,
        in_specs=[pl.BlockSpec((br, f), lambda i: (i, 0)),
                  pl.BlockSpec((1, f), lambda i: (0, 0)),
                  pl.BlockSpec((1, f), lambda i: (0, 0))],
        out_specs=pl.BlockSpec((br, f), lambda i: (i, 0)),
        out_shape=jax.ShapeDtypeStruct((n, f), jnp.float32),
    )(h, scale, shift)


def kernel(x0, x1, x2, x3, p0, p1, p2, p3, b0, b1, b2, b3,
           l3w1, l3b1, l3g1, l3be1, l3w2, l3b2, l3g2, l3be2,
           l2w1, l2b1, l2g1, l2be1, l2w2, l2b2, l2g2, l2be2,
           l1w1, l1b1, l1g1, l1be1, l1w2, l1b2, l1g2, l1be2):
    b0c = b0.reshape(-1, 1)
    b1c = b1.reshape(-1, 1)
    b2c = b2.reshape(-1, 1)
    b1r = b1.reshape(1, -1)
    b2r = b2.reshape(1, -1)
    b3r = b3.reshape(1, -1)
    p1t = p1.T
    p2t = p2.T
    p3t = p3.T

    h = _interp(x3, p3t, p2, b3r, b2c, k=1, cross_bf16=True)
    h = jnp.concatenate([h, x2], axis=1)
    h, s, ss = _linear(h, l3w1, l3b1.reshape(1, -1))
    a = _affine(s, ss, l3g1, l3be1, x2.shape[0])
    h, s, ss = _linear(h, l3w2, l3b2.reshape(1, -1), affine=a)
    a = _affine(s, ss, l3g2, l3be2, x2.shape[0])

    h = _interp(h, p2t, p1, b2r, b1c, k=3, affine=a, cross_bf16=True)
    h = jnp.concatenate([h, x1], axis=1)
    h, s, ss = _linear(h, l2w1, l2b1.reshape(1, -1))
    a = _affine(s, ss, l2g1, l2be1, x1.shape[0])
    h, s, ss = _linear(h, l2w2, l2b2.reshape(1, -1), affine=a)
    a = _affine(s, ss, l2g2, l2be2, x1.shape[0])

    h = _interp(h, p1t, p0, b1r, b0c, k=3, affine=a, cross_bf16=True)
    h = jnp.concatenate([h, x0], axis=1)
    h, s, ss = _linear(h, l1w1, l1b1.reshape(1, -1))
    a = _affine(s, ss, l1g1, l1be1, x0.shape[0])
    h, s, ss = _linear(h, l1w2, l1b2.reshape(1, -1), affine=a)
    a = _affine(s, ss, l1g2, l1be2, x0.shape[0])

    return _bn_apply(h, a)


# speedup vs baseline: 9.2761x; 1.1181x over previous
import functools

import jax
import jax.numpy as jnp
from jax.experimental import pallas as pl


def _coeffs(s_ref, ss_ref, g_ref, be_ref, n):
    mean = s_ref[:] * (1.0 / n)
    var = ss_ref[:] * (1.0 / n) - mean * mean
    scale = g_ref[:] / jnp.sqrt(var + 1e-5)
    shift = be_ref[:] - mean * scale
    return scale, shift


def _interp_body(bq_ref, q_ref, bs_ref, pt_ref, x_ref, *rest, k, n_src,
                 affine, n_prev):
    if affine:
        s_ref, ss_ref, g_ref, be_ref, o_ref = rest
    else:
        (o_ref,) = rest
    q = q_ref[:]
    q0, q1, q2 = q[:, 0:1], q[:, 1:2], q[:, 2:3]
    p0 = pt_ref[0:1, :]
    p1 = pt_ref[1:2, :]
    p2 = pt_ref[2:3, :]
    qn = q0 * q0 + q1 * q1 + q2 * q2
    pn = p0 * p0 + p1 * p1 + p2 * p2
    cross = jnp.dot(q.astype(jnp.bfloat16), pt_ref[:].astype(jnp.bfloat16),
                    preferred_element_type=jnp.float32)
    d2 = (qn - 2.0 * cross) + pn
    bmask = bq_ref[:] != bs_ref[:]
    d2 = jnp.where(bmask, 1e10, d2)

    iota = jax.lax.broadcasted_iota(jnp.int32, d2.shape, 1)
    S = None
    wsum = None
    for j in range(k):
        m = jnp.min(d2, axis=1, keepdims=True)
        am = jnp.min(jnp.where(d2 == m, iota, n_src), axis=1, keepdims=True)
        sel = iota == am
        if k == 1:
            S = sel.astype(jnp.float32)
        else:
            w = 1.0 / jnp.clip(m, 1e-16, None)
            contrib = jnp.where(sel, w, 0.0)
            S = contrib if S is None else S + contrib
            wsum = w if wsum is None else wsum + w
            if j + 1 < k:
                d2 = jnp.where(sel, 1e30, d2)

    out = jnp.dot(S, x_ref[:], preferred_element_type=jnp.float32)
    if k > 1:
        out = out / wsum
    if affine:
        scale, shift = _coeffs(s_ref, ss_ref, g_ref, be_ref, n_prev)
        out = out * scale + shift
    o_ref[:] = out.astype(o_ref.dtype)


def _interp(x, pos_src_t, pos_q, bs_row, bq_col, k, stats=None, n_prev=None,
            bq=256):
    n_src = pos_src_t.shape[1]
    n_q = pos_q.shape[0]
    c = x.shape[1]
    bq = min(bq, n_q)
    grid = (n_q // bq,)
    in_specs = [
        pl.BlockSpec((bq, 1), lambda i: (i, 0)),
        pl.BlockSpec((bq, 3), lambda i: (i, 0)),
        pl.BlockSpec((1, n_src), lambda i: (0, 0)),
        pl.BlockSpec((3, n_src), lambda i: (0, 0)),
        pl.BlockSpec((n_src, c), lambda i: (0, 0)),
    ]
    args = [bq_col, pos_q, bs_row, pos_src_t, x]
    if stats is not None:
        cs = pl.BlockSpec((1, c), lambda i: (0, 0))
        in_specs += [cs, cs, cs, cs]
        s, ss, g, be = stats
        args += [s, ss, g.reshape(1, c), be.reshape(1, c)]
    return pl.pallas_call(
        functools.partial(_interp_body, k=k, n_src=n_src,
                          affine=stats is not None,
                          n_prev=float(n_prev) if n_prev else 1.0),
        grid=grid,
        in_specs=in_specs,
        out_specs=pl.BlockSpec((bq, c), lambda i: (i, 0)),
        out_shape=jax.ShapeDtypeStruct((n_q, c), jnp.bfloat16),
    )(*args)


def _linear_body(x_ref, w_ref, b_ref, *rest, affine, skip, kx, n_prev):
    rest = list(rest)
    xs_ref = rest.pop(0) if skip else None
    if affine:
        s_ref, ss_ref, g_ref, be_ref = rest[:4]
        rest = rest[4:]
    h_ref, so_ref, sso_ref = rest
    x = x_ref[:]
    if affine:
        scale, shift = _coeffs(s_ref, ss_ref, g_ref, be_ref, n_prev)
        x = x * scale + shift
    w_head = w_ref[0:kx, :]
    if x.dtype == jnp.bfloat16:
        w_head = w_head.astype(jnp.bfloat16)
    h = jnp.dot(x, w_head, preferred_element_type=jnp.float32)
    if skip:
        h = h + jnp.dot(xs_ref[:], w_ref[kx:, :],
                        preferred_element_type=jnp.float32)
    h = h + b_ref[:]
    h = jnp.maximum(h, 0.0)
    h_ref[:] = h
    csum = jnp.sum(h, axis=0, keepdims=True)
    csq = jnp.sum(h * h, axis=0, keepdims=True)

    @pl.when(pl.program_id(0) == 0)
    def _init():
        so_ref[:] = csum
        sso_ref[:] = csq

    @pl.when(pl.program_id(0) > 0)
    def _acc():
        so_ref[:] += csum
        sso_ref[:] += csq


def _linear(x, w, b_row, skip=None, stats=None, n_prev=None, br=512):
    n, kx = x.shape
    fout = w.shape[1]
    br = min(br, n)
    grid = (n // br,)
    in_specs = [
        pl.BlockSpec((br, kx), lambda i: (i, 0)),
        pl.BlockSpec(w.shape, lambda i: (0, 0)),
        pl.BlockSpec((1, fout), lambda i: (0, 0)),
    ]
    args = [x, w, b_row]
    if skip is not None:
        ks = skip.shape[1]
        in_specs.append(pl.BlockSpec((br, ks), lambda i: (i, 0)))
        args.append(skip)
    if stats is not None:
        cs = pl.BlockSpec((1, kx), lambda i: (0, 0))
        in_specs += [cs, cs, cs, cs]
        s, ss, g, be = stats
        args += [s, ss, g.reshape(1, kx), be.reshape(1, kx)]
    return pl.pallas_call(
        functools.partial(_linear_body, affine=stats is not None,
                          skip=skip is not None, kx=kx,
                          n_prev=float(n_prev) if n_prev else 1.0),
        grid=grid,
        in_specs=in_specs,
        out_specs=[
            pl.BlockSpec((br, fout), lambda i: (i, 0)),
            pl.BlockSpec((1, fout), lambda i: (0, 0)),
            pl.BlockSpec((1, fout), lambda i: (0, 0)),
        ],
        out_shape=[
            jax.ShapeDtypeStruct((n, fout), jnp.float32),
            jax.ShapeDtypeStruct((1, fout), jnp.float32),
            jax.ShapeDtypeStruct((1, fout), jnp.float32),
        ],
    )(*args)


def _bn_apply_body(h_ref, s_ref, ss_ref, g_ref, be_ref, o_ref, *, n_prev):
    scale, shift = _coeffs(s_ref, ss_ref, g_ref, be_ref, n_prev)
    o_ref[:] = h_ref[:] * scale + shift


def _bn_apply(h, stats, n_prev, br=2048):
    n, f = h.shape
    s, ss, g, be = stats
    br = min(br, n)
    grid = (n // br,)
    cs = pl.BlockSpec((1, f), lambda i: (0, 0))
    return pl.pallas_call(
        functools.partial(_bn_apply_body, n_prev=float(n_prev)),
        grid=grid,
        in_specs=[pl.BlockSpec((br, f), lambda i: (i, 0)), cs, cs, cs, cs],
        out_specs=pl.BlockSpec((br, f), lambda i: (i, 0)),
        out_shape=jax.ShapeDtypeStruct((n, f), jnp.float32),
    )(h, s, ss, g.reshape(1, f), be.reshape(1, f))


def kernel(x0, x1, x2, x3, p0, p1, p2, p3, b0, b1, b2, b3,
           l3w1, l3b1, l3g1, l3be1, l3w2, l3b2, l3g2, l3be2,
           l2w1, l2b1, l2g1, l2be1, l2w2, l2b2, l2g2, l2be2,
           l1w1, l1b1, l1g1, l1be1, l1w2, l1b2, l1g2, l1be2):
    n0, n1, n2 = x0.shape[0], x1.shape[0], x2.shape[0]
    b0c = b0.reshape(-1, 1)
    b1c = b1.reshape(-1, 1)
    b2c = b2.reshape(-1, 1)
    b1r = b1.reshape(1, -1)
    b2r = b2.reshape(1, -1)
    b3r = b3.reshape(1, -1)

    h = _interp(x3, p3.T, p2, b3r, b2c, k=1)
    h, s, ss = _linear(h, l3w1, l3b1.reshape(1, -1), skip=x2)
    h, s, ss = _linear(h, l3w2, l3b2.reshape(1, -1),
                       stats=(s, ss, l3g1, l3be1), n_prev=n2)

    h = _interp(h, p2.T, p1, b2r, b1c, k=3,
                stats=(s, ss, l3g2, l3be2), n_prev=n2)
    h, s, ss = _linear(h, l2w1, l2b1.reshape(1, -1), skip=x1)
    h, s, ss = _linear(h, l2w2, l2b2.reshape(1, -1),
                       stats=(s, ss, l2g1, l2be1), n_prev=n1)

    h = _interp(h, p1.T, p0, b1r, b0c, k=3,
                stats=(s, ss, l2g2, l2be2), n_prev=n1)
    h, s, ss = _linear(h, l1w1, l1b1.reshape(1, -1), skip=x0)
    h, s, ss = _linear(h, l1w2, l1b2.reshape(1, -1),
                       stats=(s, ss, l1g1, l1be1), n_prev=n0)

    return _bn_apply(h, (s, ss, l1g2, l1be2), n0)
